# baseline (speedup 1.0000x reference)
import functools

import jax
import jax.numpy as jnp
from jax import lax
from jax.experimental import pallas as pl
from jax.experimental.pallas import tpu as pltpu
from jax.experimental.pallas import tpu_sc as plsc

N = 8192
LANES = 16
NCH = N // LANES


def _sc_body(left_hbm, right_hbm, reroll_hbm,
             mask_out, score_out, reroll_out, cnt_out,
             left_v, right_v, aux_v, buf2_v, work_v, marker_v, cnt_v):
    c = lax.axis_index("c")
    s = lax.axis_index("s")

    @pl.when((s == 0) & (c == 0))
    def _reachability():
        pltpu.sync_copy(left_hbm, left_v)
        pltpu.sync_copy(right_hbm, right_v)

        def zero_chunk(i, carry):
            work_v[pl.ds(i * LANES, LANES)] = jnp.zeros((LANES,), jnp.int32)
            return carry

        lax.fori_loop(0, NCH, zero_chunk, 0)
        lanes = lax.broadcasted_iota(jnp.int32, (LANES,), 0)
        work_v[pl.ds(0, LANES)] = (lanes == 0).astype(jnp.int32)

        aux_v[pl.ds(0, LANES)] = jnp.zeros((LANES,), jnp.int32)
        ones = jnp.ones((LANES,), jnp.int32)

        def level(carry):
            total, cnt_f = carry

            def fvec(k, cnt_new):
                valid = lanes < (cnt_f - k * LANES)
                nodes = aux_v[pl.ds(k * LANES, LANES)]
                nodes = jnp.where(valid, nodes, 0)
                lv = plsc.load_gather(left_v, [nodes], mask=valid)
                rv = plsc.load_gather(right_v, [nodes], mask=valid)

                def expand(child, cnt):
                    m = valid & (child >= 0)
                    cc = jnp.where(m, child, 0)
                    g = plsc.load_gather(work_v, [cc], mask=m)
                    cand = m & (g == 0)
                    plsc.store_scatter(marker_v, [cc], lanes, mask=cand)
                    gm = plsc.load_gather(marker_v, [cc], mask=cand)
                    win = cand & (gm == lanes)
                    plsc.store_scatter(work_v, [cc], ones, mask=win)
                    wi = win.astype(jnp.int32)
                    pos = cnt + plsc.cumsum(wi) - 1
                    pos = jnp.where(win, pos, 0)
                    plsc.store_scatter(buf2_v, [pos], cc, mask=win)
                    return cnt + jnp.sum(wi)

                cnt_new = expand(lv, cnt_new)
                cnt_new = expand(rv, cnt_new)
                return cnt_new

            nvec = (cnt_f + LANES - 1) // LANES
            cnt_new = lax.fori_loop(0, nvec, fvec, jnp.int32(0))

            def cpy(k, carry2):
                aux_v[pl.ds(k * LANES, LANES)] = buf2_v[pl.ds(k * LANES, LANES)]
                return carry2

            lax.fori_loop(0, (cnt_new + LANES - 1) // LANES, cpy, 0)
            return (total + cnt_new, cnt_new)

        final_cnt, _ = lax.while_loop(
            lambda tc: tc[1] > 0, level, (jnp.int32(1), jnp.int32(1)))

        cnt_v[...] = jnp.full((LANES,), final_cnt, jnp.int32)
        pltpu.sync_copy(work_v.at[pl.ds(0, N)], mask_out)
        pltpu.sync_copy(cnt_v, cnt_out)

    @pl.when((s == 1) & (c == 0))
    def _leaf_flags():
        pltpu.sync_copy(left_hbm, left_v)
        pltpu.sync_copy(right_hbm, right_v)
        pltpu.sync_copy(reroll_hbm, work_v.at[pl.ds(N, N)])

        def sweep(t, carry):
            i = NCH - 1 - t
            base = i * LANES
            nid = base + lax.broadcasted_iota(jnp.int32, (LANES,), 0)
            lv = left_v[pl.ds(base, LANES)]
            rv = right_v[pl.ds(base, LANES)]
            work_v[pl.ds(base, LANES)] = jnp.zeros((LANES,), jnp.int32)

            ml = (lv < 0) | (lv > nid)
            mr = (rv < 0) | (rv > nid)
            il = jnp.where(ml, jnp.where(lv < 0, (N - 1) - lv, lv), 0)
            ir = jnp.where(mr, jnp.where(rv < 0, (N - 1) - rv, rv), 0)

            def contrib():
                gl = plsc.load_gather(work_v, [il], mask=ml)
                gr = plsc.load_gather(work_v, [ir], mask=mr)
                gl = jnp.where(ml, gl, 0)
                gr = jnp.where(mr, gr, 0)
                cl = jnp.where(lv < 0, gl + 1, gl)
                cr = jnp.where(rv < 0, gr + 1, gr)
                return cl | cr

            work_v[pl.ds(base, LANES)] = contrib()

            inchunk = ((lv > nid) & (lv < base + LANES)) | \
                      ((rv > nid) & (rv < base + LANES))

            @pl.when(jnp.any(inchunk))
            def _fixpoint():
                def upd(_):
                    v = work_v[pl.ds(base, LANES)]
                    nv = contrib()
                    work_v[pl.ds(base, LANES)] = nv
                    return jnp.any(nv != v)

                lax.while_loop(lambda ch: ch, upd, jnp.bool_(True))

            v = work_v[pl.ds(base, LANES)]
            aux_v[pl.ds(base, LANES)] = v & 1
            buf2_v[pl.ds(base, LANES)] = (v >> 1) & 1
            return carry

        lax.fori_loop(0, NCH, sweep, 0)
        pltpu.sync_copy(aux_v, score_out)
        pltpu.sync_copy(buf2_v, reroll_out)


@jax.jit
def _dag_flags(left, right, reroll_i32):
    mesh = plsc.VectorSubcoreMesh(core_axis_name="c", subcore_axis_name="s",
                                  num_cores=1)
    f = pl.kernel(
        _sc_body,
        out_type=(
            jax.ShapeDtypeStruct((N,), jnp.int32),
            jax.ShapeDtypeStruct((N,), jnp.int32),
            jax.ShapeDtypeStruct((N,), jnp.int32),
            jax.ShapeDtypeStruct((LANES,), jnp.int32),
        ),
        mesh=mesh,
        compiler_params=pltpu.CompilerParams(needs_layout_passes=False),
        scratch_types=(
            pltpu.VMEM((N,), jnp.int32),
            pltpu.VMEM((N,), jnp.int32),
            pltpu.VMEM((N,), jnp.int32),
            pltpu.VMEM((N,), jnp.int32),
            pltpu.VMEM((2 * N,), jnp.int32),
            pltpu.VMEM((N,), jnp.int32),
            pltpu.VMEM((LANES,), jnp.int32),
        ),
    )
    return f(left, right, reroll_i32)


def kernel(thresholds, rules, binary_ops, left, right, leaf_is_reroll,
           leaf_mask_left, leaf_mask_right, leaf_mask_op, leaf_score_cat):
    mask_i, score_i, reroll_i, cnt = _dag_flags(
        left, right, leaf_is_reroll.astype(jnp.int32))
    return (mask_i.astype(jnp.bool_), score_i.astype(jnp.bool_),
            reroll_i.astype(jnp.bool_), cnt[0])


# speedup vs baseline: 1.0035x; 1.0035x over previous
import functools

import jax
import jax.numpy as jnp
from jax import lax
from jax.experimental import pallas as pl
from jax.experimental.pallas import tpu as pltpu
from jax.experimental.pallas import tpu_sc as plsc

N = 8192
LANES = 16
NCH = N // LANES


def _sc_body(left_hbm, right_hbm, reroll_hbm,
             mask_out, score_out, reroll_out, cnt_out,
             left_v, right_v, aux_v, buf2_v, work_v, marker_v, cnt_v):
    c = lax.axis_index("c")
    s = lax.axis_index("s")

    @pl.when((s == 0) & (c == 0))
    def _reachability():
        pltpu.sync_copy(left_hbm, left_v)
        pltpu.sync_copy(right_hbm, right_v)

        def zero_chunk(i, carry):
            work_v[pl.ds(i * LANES, LANES)] = jnp.zeros((LANES,), jnp.int32)
            return carry

        lax.fori_loop(0, NCH, zero_chunk, 0)
        lanes = lax.broadcasted_iota(jnp.int32, (LANES,), 0)
        work_v[pl.ds(0, LANES)] = (lanes == 0).astype(jnp.int32)

        aux_v[pl.ds(0, LANES)] = jnp.zeros((LANES,), jnp.int32)
        ones = jnp.ones((LANES,), jnp.int32)

        def level(carry):
            total, cnt_f = carry

            def fvec(k, cnt_new):
                valid = lanes < (cnt_f - k * LANES)
                nodes = aux_v[pl.ds(k * LANES, LANES)]
                nodes = jnp.where(valid, nodes, 0)
                lv = plsc.load_gather(left_v, [nodes], mask=valid)
                rv = plsc.load_gather(right_v, [nodes], mask=valid)

                def expand(child, cnt):
                    m = valid & (child >= 0)
                    cc = jnp.where(m, child, 0)
                    g = plsc.load_gather(work_v, [cc], mask=m)
                    cand = m & (g == 0)
                    plsc.store_scatter(marker_v, [cc], lanes, mask=cand)
                    gm = plsc.load_gather(marker_v, [cc], mask=cand)
                    win = cand & (gm == lanes)
                    plsc.store_scatter(work_v, [cc], ones, mask=win)
                    wi = win.astype(jnp.int32)
                    pos = cnt + plsc.cumsum(wi) - 1
                    pos = jnp.where(win, pos, 0)
                    plsc.store_scatter(buf2_v, [pos], cc, mask=win)
                    return cnt + jnp.sum(wi)

                cnt_new = expand(lv, cnt_new)
                cnt_new = expand(rv, cnt_new)
                return cnt_new

            nvec = (cnt_f + LANES - 1) // LANES
            cnt_new = lax.fori_loop(0, nvec, fvec, jnp.int32(0))

            def cpy(k, carry2):
                aux_v[pl.ds(k * LANES, LANES)] = buf2_v[pl.ds(k * LANES, LANES)]
                return carry2

            lax.fori_loop(0, (cnt_new + LANES - 1) // LANES, cpy, 0)
            return (total + cnt_new, cnt_new)

        final_cnt, _ = lax.while_loop(
            lambda tc: tc[1] > 0, level, (jnp.int32(1), jnp.int32(1)))

        cnt_v[...] = jnp.full((LANES,), final_cnt, jnp.int32)
        pltpu.sync_copy(work_v.at[pl.ds(0, N)], mask_out)
        pltpu.sync_copy(cnt_v, cnt_out)

    @pl.when((s == 1) & (c == 0))
    def _leaf_flags():
        pltpu.sync_copy(left_hbm, left_v)
        pltpu.sync_copy(right_hbm, right_v)
        pltpu.sync_copy(reroll_hbm, work_v.at[pl.ds(N, N)])

        def sweep(t, carry):
            i = NCH - 1 - t
            base = i * LANES
            nid = base + lax.broadcasted_iota(jnp.int32, (LANES,), 0)
            lv = left_v[pl.ds(base, LANES)]
            rv = right_v[pl.ds(base, LANES)]
            work_v[pl.ds(base, LANES)] = jnp.zeros((LANES,), jnp.int32)

            ml = (lv < 0) | (lv > nid)
            mr = (rv < 0) | (rv > nid)
            il = jnp.where(ml, jnp.where(lv < 0, (N - 1) - lv, lv), 0)
            ir = jnp.where(mr, jnp.where(rv < 0, (N - 1) - rv, rv), 0)

            def contrib():
                gl = plsc.load_gather(work_v, [il], mask=ml)
                gr = plsc.load_gather(work_v, [ir], mask=mr)
                gl = jnp.where(ml, gl, 0)
                gr = jnp.where(mr, gr, 0)
                cl = jnp.where(lv < 0, gl + 1, gl)
                cr = jnp.where(rv < 0, gr + 1, gr)
                return cl | cr

            work_v[pl.ds(base, LANES)] = contrib()

            inchunk = ((lv > nid) & (lv < base + LANES)) | \
                      ((rv > nid) & (rv < base + LANES))

            @pl.when(jnp.any(inchunk))
            def _fixpoint():
                def upd(_):
                    v = work_v[pl.ds(base, LANES)]
                    nv = contrib()
                    work_v[pl.ds(base, LANES)] = nv
                    return jnp.any(nv != v)

                lax.while_loop(lambda ch: ch, upd, jnp.bool_(True))

            v = work_v[pl.ds(base, LANES)]
            aux_v[pl.ds(base, LANES)] = v & 1
            buf2_v[pl.ds(base, LANES)] = (v >> 1) & 1
            return carry

        lax.fori_loop(0, NCH, sweep, 0)
        pltpu.sync_copy(aux_v, score_out)
        pltpu.sync_copy(buf2_v, reroll_out)


@jax.jit
def _dag_flags(left, right, reroll_i32):
    mesh = plsc.VectorSubcoreMesh(core_axis_name="c", subcore_axis_name="s",
                                  num_cores=1)
    f = pl.kernel(
        _sc_body,
        out_type=(
            jax.ShapeDtypeStruct((N,), jnp.int32),
            jax.ShapeDtypeStruct((N,), jnp.int32),
            jax.ShapeDtypeStruct((N,), jnp.int32),
            jax.ShapeDtypeStruct((LANES,), jnp.int32),
        ),
        mesh=mesh,
        compiler_params=pltpu.CompilerParams(
            needs_layout_passes=False,
            disable_bounds_checks=True,
            skip_device_barrier=True,
        ),
        scratch_types=(
            pltpu.VMEM((N,), jnp.int32),
            pltpu.VMEM((N,), jnp.int32),
            pltpu.VMEM((N,), jnp.int32),
            pltpu.VMEM((N,), jnp.int32),
            pltpu.VMEM((2 * N,), jnp.int32),
            pltpu.VMEM((N,), jnp.int32),
            pltpu.VMEM((LANES,), jnp.int32),
        ),
    )
    return f(left, right, reroll_i32)


def kernel(thresholds, rules, binary_ops, left, right, leaf_is_reroll,
           leaf_mask_left, leaf_mask_right, leaf_mask_op, leaf_score_cat):
    mask_i, score_i, reroll_i, cnt = _dag_flags(
        left, right, leaf_is_reroll.astype(jnp.int32))
    return (mask_i.astype(jnp.bool_), score_i.astype(jnp.bool_),
            reroll_i.astype(jnp.bool_), cnt[0])


# speedup vs baseline: 1.0260x; 1.0224x over previous
import functools

import jax
import jax.numpy as jnp
from jax import lax
from jax.experimental import pallas as pl
from jax.experimental.pallas import tpu as pltpu
from jax.experimental.pallas import tpu_sc as plsc

N = 8192
LANES = 16
NCH = N // LANES


def _sc_body(left_hbm, right_hbm, reroll_hbm,
             mask_out, score_out, reroll_out, cnt_out,
             left_v, right_v, aux_v, buf2_v, work_v, marker_v, cnt_v):
    c = lax.axis_index("c")
    s = lax.axis_index("s")

    @pl.when((s == 0) & (c == 0))
    def _reachability():
        pltpu.sync_copy(left_hbm, left_v)
        pltpu.sync_copy(right_hbm, right_v)

        def zero_chunk(i, carry):
            work_v[pl.ds(i * LANES, LANES)] = jnp.zeros((LANES,), jnp.int32)
            return carry

        lax.fori_loop(0, NCH, zero_chunk, 0)
        lanes = lax.broadcasted_iota(jnp.int32, (LANES,), 0)
        work_v[pl.ds(0, LANES)] = (lanes == 0).astype(jnp.int32)

        aux_v[pl.ds(0, LANES)] = jnp.zeros((LANES,), jnp.int32)
        ones = jnp.ones((LANES,), jnp.int32)

        def level(carry):
            total, cnt_f = carry

            def fvec(k, cnt_new):
                valid = lanes < (cnt_f - k * LANES)
                nodes = aux_v[pl.ds(k * LANES, LANES)]
                nodes = jnp.where(valid, nodes, 0)
                lv = plsc.load_gather(left_v, [nodes], mask=valid)
                rv = plsc.load_gather(right_v, [nodes], mask=valid)

                def expand(child, cnt):
                    m = valid & (child >= 0)
                    cc = jnp.where(m, child, 0)
                    g = plsc.load_gather(work_v, [cc], mask=m)
                    cand = m & (g == 0)
                    plsc.store_scatter(marker_v, [cc], lanes, mask=cand)
                    gm = plsc.load_gather(marker_v, [cc], mask=cand)
                    win = cand & (gm == lanes)
                    plsc.store_scatter(work_v, [cc], ones, mask=win)
                    wi = win.astype(jnp.int32)
                    pos = cnt + plsc.cumsum(wi) - 1
                    pos = jnp.where(win, pos, 0)
                    plsc.store_scatter(buf2_v, [pos], cc, mask=win)
                    return cnt + jnp.sum(wi)

                cnt_new = expand(lv, cnt_new)
                cnt_new = expand(rv, cnt_new)
                return cnt_new

            nvec = (cnt_f + LANES - 1) // LANES
            cnt_new = lax.fori_loop(0, nvec, fvec, jnp.int32(0))

            def cpy(k, carry2):
                aux_v[pl.ds(k * LANES, LANES)] = buf2_v[pl.ds(k * LANES, LANES)]
                return carry2

            lax.fori_loop(0, (cnt_new + LANES - 1) // LANES, cpy, 0)
            return (total + cnt_new, cnt_new)

        final_cnt, _ = lax.while_loop(
            lambda tc: tc[1] > 0, level, (jnp.int32(1), jnp.int32(1)))

        cnt_v[...] = jnp.full((LANES,), final_cnt, jnp.int32)
        pltpu.sync_copy(work_v.at[pl.ds(0, N)], mask_out)
        pltpu.sync_copy(cnt_v, cnt_out)

    @pl.when((s == 1) & (c == 0))
    def _leaf_flags():
        pltpu.sync_copy(left_hbm, left_v)
        pltpu.sync_copy(right_hbm, right_v)
        pltpu.sync_copy(reroll_hbm, work_v.at[pl.ds(N, N)])

        def do_chunk(i):
            base = i * LANES
            nid = base + lax.broadcasted_iota(jnp.int32, (LANES,), 0)
            lv = left_v[pl.ds(base, LANES)]
            rv = right_v[pl.ds(base, LANES)]
            work_v[pl.ds(base, LANES)] = jnp.zeros((LANES,), jnp.int32)

            ml = (lv < 0) | (lv > nid)
            mr = (rv < 0) | (rv > nid)
            il = jnp.where(ml, jnp.where(lv < 0, (N - 1) - lv, lv), 0)
            ir = jnp.where(mr, jnp.where(rv < 0, (N - 1) - rv, rv), 0)

            def contrib():
                gl = plsc.load_gather(work_v, [il], mask=ml)
                gr = plsc.load_gather(work_v, [ir], mask=mr)
                gl = jnp.where(ml, gl, 0)
                gr = jnp.where(mr, gr, 0)
                cl = jnp.where(lv < 0, gl + 1, gl)
                cr = jnp.where(rv < 0, gr + 1, gr)
                return cl | cr

            work_v[pl.ds(base, LANES)] = contrib()

            inchunk = ((lv > nid) & (lv < base + LANES)) | \
                      ((rv > nid) & (rv < base + LANES))

            @pl.when(jnp.any(inchunk))
            def _fixpoint():
                def upd(_):
                    v = work_v[pl.ds(base, LANES)]
                    nv = contrib()
                    work_v[pl.ds(base, LANES)] = nv
                    return jnp.any(nv != v)

                lax.while_loop(lambda ch: ch, upd, jnp.bool_(True))

            v = work_v[pl.ds(base, LANES)]
            aux_v[pl.ds(base, LANES)] = v & 1
            buf2_v[pl.ds(base, LANES)] = (v >> 1) & 1

        def sweep(t, carry):
            do_chunk(NCH - 1 - 2 * t)
            do_chunk(NCH - 2 - 2 * t)
            return carry

        lax.fori_loop(0, NCH // 2, sweep, 0)
        pltpu.sync_copy(aux_v, score_out)
        pltpu.sync_copy(buf2_v, reroll_out)


@jax.jit
def _dag_flags(left, right, reroll_i32):
    mesh = plsc.VectorSubcoreMesh(core_axis_name="c", subcore_axis_name="s",
                                  num_cores=1)
    f = pl.kernel(
        _sc_body,
        out_type=(
            jax.ShapeDtypeStruct((N,), jnp.int32),
            jax.ShapeDtypeStruct((N,), jnp.int32),
            jax.ShapeDtypeStruct((N,), jnp.int32),
            jax.ShapeDtypeStruct((LANES,), jnp.int32),
        ),
        mesh=mesh,
        compiler_params=pltpu.CompilerParams(
            needs_layout_passes=False,
            disable_bounds_checks=True,
            skip_device_barrier=True,
        ),
        scratch_types=(
            pltpu.VMEM((N,), jnp.int32),
            pltpu.VMEM((N,), jnp.int32),
            pltpu.VMEM((N,), jnp.int32),
            pltpu.VMEM((N,), jnp.int32),
            pltpu.VMEM((2 * N,), jnp.int32),
            pltpu.VMEM((N,), jnp.int32),
            pltpu.VMEM((LANES,), jnp.int32),
        ),
    )
    return f(left, right, reroll_i32)


def kernel(thresholds, rules, binary_ops, left, right, leaf_is_reroll,
           leaf_mask_left, leaf_mask_right, leaf_mask_op, leaf_score_cat):
    mask_i, score_i, reroll_i, cnt = _dag_flags(
        left, right, leaf_is_reroll.astype(jnp.int32))
    return (mask_i.astype(jnp.bool_), score_i.astype(jnp.bool_),
            reroll_i.astype(jnp.bool_), cnt[0])
